# baseline (speedup 1.0000x reference)
import functools

import jax
import jax.numpy as jnp
from jax import lax
from jax.experimental import pallas as pl
from jax.experimental.pallas import tpu as pltpu
from jax.experimental.pallas import tpu_sc as plsc

_SHAPE = (4096, 16384)
_N = _SHAPE[0] * _SHAPE[1]
_K = 671088

_SW = 1 << 20
_NWIN = _N // _SW
_TS = _SW // 16
_CP = 512
_CB = 528

_mesh = plsc.VectorSubcoreMesh(core_axis_name="c", subcore_axis_name="s")


@functools.partial(
    pl.kernel,
    mesh=_mesh,
    compiler_params=pltpu.CompilerParams(needs_layout_passes=False),
    out_type=jax.ShapeDtypeStruct((_N,), jnp.float32),
    scratch_types=[
        pltpu.VMEM_SHARED((_SW,), jnp.float32),
        pltpu.VMEM((_CB,), jnp.int32),
        pltpu.VMEM((_CB,), jnp.float32),
        pltpu.VMEM((_CB,), jnp.int32),
        pltpu.VMEM((_CB,), jnp.float32),
        pltpu.VMEM((80,), jnp.int32),
        pltpu.SemaphoreType.DMA,
        pltpu.SemaphoreType.DMA,
        pltpu.SemaphoreType.DMA,
    ],
)
def _sc_apply(tensor_hbm, idx_hbm, val_hbm, bounds_hbm, out_hbm,
              spmem, ibuf, vbuf, istg, vstg, bounds_v, lsem, csem, ssem):
    c = lax.axis_index("c")
    s = lax.axis_index("s")
    pltpu.sync_copy(bounds_hbm, bounds_v)
    iota = lax.iota(jnp.int32, 16)
    lane16 = jnp.full((16,), 16, dtype=jnp.int32)

    def bound_at(i):
        return jnp.max(plsc.load_gather(bounds_v,
                                        [jnp.full((16,), i, jnp.int32)]))

    @pl.loop(0, _NWIN // 2)
    def _win(wi):
        w = wi * 2 + c
        wlo = w * _SW
        off = s * _TS

        pltpu.async_copy(tensor_hbm.at[pl.ds(wlo + off, _TS)],
                         spmem.at[pl.ds(off, _TS)], lsem).wait()
        plsc.subcore_barrier()

        s_lo = bound_at(w)
        s_hi = bound_at(w + 1)
        n_ch = lax.div(s_hi - s_lo + (_CP - 1), _CP)
        n_mine = jnp.maximum(lax.div(n_ch - s + 15, 16), 0)

        @pl.loop(0, n_mine)
        def _chunk(it):
            g = s_lo + (s + it * 16) * _CP
            a8 = pl.multiple_of(
                jnp.minimum(g - lax.rem(g, 8), _K - _CB), 8)
            pltpu.sync_copy(idx_hbm.at[pl.ds(a8, _CB)], ibuf)
            pltpu.sync_copy(val_hbm.at[pl.ds(a8, _CB)], vbuf)
            for gi in range(_CB // 16):
                sl = pl.ds(gi * 16, 16)
                pos = (a8 + gi * 16) + iota
                iv = ibuf[sl]
                vv = vbuf[sl]
                valid = (pos >= g) & (pos < g + _CP) & (pos < s_hi)
                istg[sl] = jnp.where(valid, iv - wlo,
                                     (gi * 16) * lane16 + iota * 16)
                vstg[sl] = jnp.where(valid, vv, jnp.zeros((16,),
                                                          jnp.float32))
            pltpu.async_copy(vstg, spmem.at[istg], csem, add=True).wait()

        plsc.subcore_barrier()

        pltpu.async_copy(spmem.at[pl.ds(off, _TS)],
                         out_hbm.at[pl.ds(wlo + off, _TS)], ssem).wait()
        plsc.subcore_barrier()


def kernel(tensor, values, indices):
    flat = tensor.reshape(-1)
    idx = indices.astype(jnp.int32)
    edges = jnp.arange(0, _N + 1, _SW, dtype=jnp.int32)
    bounds = jnp.searchsorted(idx, edges).astype(jnp.int32)
    bounds = jnp.pad(bounds, (0, 80 - bounds.shape[0]))
    out = _sc_apply(flat, idx, values, bounds)
    return out.reshape(_SHAPE)
